# baseline (device time: 26209 ns/iter reference)
import jax
import jax.numpy as jnp
from jax import lax
from jax.experimental import pallas as pl
from jax.experimental.pallas import tpu as pltpu

N_DEV = 4
B_PER = 2
SQ = 128
HG = 4
DH = 64
D_MODEL = 512
D_HEADS = HG * DH
HPP = 2
W_HALF = HPP * DH

_GROUP_OFF = (0, 3, 1, 2)


def _unit_static(xb, w_s, k_ref, v_ref, g, parity):
    wq = w_s[:, 0:W_HALF]
    wot = w_s[:, W_HALF:2 * W_HALF]
    q = jnp.dot(xb, wq, preferred_element_type=jnp.float32)
    ctx_rows = []
    for b in range(B_PER):
        ctx_heads = []
        for j in range(HPP):
            hh = g * HG + parity * HPP + j
            qbh = q[b * SQ:(b + 1) * SQ, j * DH:(j + 1) * DH]
            kb = k_ref[b, :, hh, :]
            sc = lax.dot_general(
                qbh, kb, (((1,), (1,)), ((), ())),
                preferred_element_type=jnp.float32,
            ) * 0.125
            sc = sc - jnp.max(sc, axis=1, keepdims=True)
            e = jnp.exp(sc)
            w = e / jnp.sum(e, axis=1, keepdims=True)
            vb = v_ref[b, :, hh, :]
            ctx_heads.append(jnp.dot(w, vb, preferred_element_type=jnp.float32))
        ctx_rows.append(jnp.concatenate(ctx_heads, axis=1))
    ctx = jnp.concatenate(ctx_rows, axis=0).astype(jnp.bfloat16)
    return lax.dot_general(
        ctx, wot, (((1,), (1,)), ((), ())),
        preferred_element_type=jnp.float32,
    )


def _body(x_ref, w_ref, k_ref, v_ref, out_ref, comm_ref, send_sems, recv_sems):
    my = lax.axis_index("i")
    right = lax.rem(my + 1, N_DEV)
    left = lax.rem(my + N_DEV - 1, N_DEV)
    groups = [lax.rem(my + off, N_DEV) for off in _GROUP_OFF]

    barrier_sem = pltpu.get_barrier_semaphore()
    for nbr in (left, right):
        pl.semaphore_signal(
            barrier_sem, inc=1,
            device_id=(nbr,), device_id_type=pl.DeviceIdType.MESH,
        )
    pl.semaphore_wait(barrier_sem, 2)

    xb = x_ref[...].astype(jnp.bfloat16)

    def unit(w_u32, g, parity, first=False):
        w_s = pltpu.bitcast(w_u32, jnp.bfloat16)
        for c in range(N_DEV):
            @pl.when(g == c)
            def _(c=c):
                part = _unit_static(xb, w_s, k_ref, v_ref, c, parity)
                if first:
                    out_ref[...] = part
                else:
                    out_ref[...] = out_ref[...] + part

    def rdma(src_ref, dst_slot, send_idx, target):
        return pltpu.make_async_remote_copy(
            src_ref=src_ref,
            dst_ref=comm_ref.at[dst_slot],
            send_sem=send_sems.at[send_idx],
            recv_sem=recv_sems.at[dst_slot],
            device_id=(target,),
            device_id_type=pl.DeviceIdType.MESH,
        )

    cw0a = rdma(w_ref.at[0], 0, 0, right)
    cw0b = rdma(w_ref.at[1], 1, 1, right)
    ccw0a = rdma(w_ref.at[1], 3, 2, left)
    ccw0b = rdma(w_ref.at[0], 2, 3, left)
    cw0a.start()
    ccw0a.start()
    cw0b.start()
    ccw0b.start()

    unit(w_ref[0], groups[0], 0, first=True)

    cw0a.wait_recv()
    fwd_cw = rdma(comm_ref.at[0], 4, 4, right)
    fwd_cw.start()
    ccw0a.wait_recv()
    fwd_ccw = rdma(comm_ref.at[3], 5, 5, left)
    fwd_ccw.start()

    unit(w_ref[1], groups[0], 1)
    unit(comm_ref[0], groups[1], 0)
    unit(comm_ref[3], groups[2], 1)

    cw0b.wait_recv()
    unit(comm_ref[1], groups[1], 1)
    ccw0b.wait_recv()
    unit(comm_ref[2], groups[2], 0)

    fwd_cw.wait_recv()
    unit(comm_ref[4], groups[3], 0)
    fwd_ccw.wait_recv()
    unit(comm_ref[5], groups[3], 1)

    for d in (cw0a, cw0b, ccw0a, ccw0b, fwd_cw, fwd_ccw):
        d.wait_send()


def kernel(x, Wq, K_ext, V_ext, Wo):
    my = lax.axis_index("i")

    kb = lax.dynamic_slice_in_dim(K_ext, my * B_PER, B_PER, axis=0)
    vb = lax.dynamic_slice_in_dim(V_ext, my * B_PER, B_PER, axis=0)

    xf = x.reshape(B_PER * SQ, D_MODEL)
    wq_h = Wq.astype(jnp.bfloat16).reshape(D_MODEL, HG // HPP, W_HALF).transpose(1, 0, 2)
    wot_h = Wo.T.astype(jnp.bfloat16).reshape(D_MODEL, HG // HPP, W_HALF).transpose(1, 0, 2)
    pack_bf = jnp.concatenate([wq_h, wot_h], axis=2)
    pack_u16 = lax.bitcast_convert_type(pack_bf, jnp.uint16)
    pack_u32 = lax.bitcast_convert_type(
        pack_u16.reshape(2, D_MODEL // 2, 2, D_HEADS).transpose(0, 1, 3, 2),
        jnp.uint32,
    )

    out = pl.pallas_call(
        _body,
        out_shape=jax.ShapeDtypeStruct((B_PER * SQ, D_MODEL), jnp.float32),
        in_specs=[
            pl.BlockSpec(memory_space=pltpu.VMEM),
            pl.BlockSpec(memory_space=pltpu.VMEM),
            pl.BlockSpec(memory_space=pltpu.VMEM),
            pl.BlockSpec(memory_space=pltpu.VMEM),
        ],
        out_specs=pl.BlockSpec(memory_space=pltpu.VMEM),
        scratch_shapes=[
            pltpu.VMEM((6, D_MODEL // 2, D_HEADS), jnp.uint32),
            pltpu.SemaphoreType.DMA((6,)),
            pltpu.SemaphoreType.DMA((6,)),
        ],
        compiler_params=pltpu.CompilerParams(collective_id=0),
    )(xf, pack_u32, kb, vb)

    return out.reshape(B_PER, SQ, D_MODEL)


# device time: 22193 ns/iter; 1.1810x vs baseline; 1.1810x over previous
import jax
import jax.numpy as jnp
from jax import lax
from jax.experimental import pallas as pl
from jax.experimental.pallas import tpu as pltpu

N_DEV = 4
B_PER = 2
SQ = 128
HG = 4
DH = 64
D_MODEL = 512
D_HEADS = HG * DH
HPP = 2
W_HALF = HPP * DH

_GROUP_OFF = (0, 3, 1, 2)


def _unit_contrib(xb, w_u32, k_ref, v_ref, g, s):
    w_s = pltpu.bitcast(w_u32, jnp.bfloat16)
    wq = w_s[:, 0:W_HALF]
    wot = w_s[:, W_HALF:2 * W_HALF]
    q = jnp.dot(xb, wq, preferred_element_type=jnp.float32)
    ctx_rows = []
    for b in range(B_PER):
        ctx_heads = []
        for j in range(HPP):
            hh = g * HG + (s % 2) * HPP + j
            qbh = q[b * SQ:(b + 1) * SQ, j * DH:(j + 1) * DH]
            kt = k_ref[b, hh]
            sc = jnp.dot(qbh, kt, preferred_element_type=jnp.float32) * 0.125
            sc = sc - jnp.max(sc, axis=1, keepdims=True)
            e = jnp.exp(sc)
            w = e / jnp.sum(e, axis=1, keepdims=True)
            vt = v_ref[b, hh]
            ctx_heads.append(lax.dot_general(
                w, vt, (((1,), (1,)), ((), ())),
                preferred_element_type=jnp.float32,
            ))
        ctx_rows.append(jnp.concatenate(ctx_heads, axis=1))
    ctx = jnp.concatenate(ctx_rows, axis=0).astype(jnp.bfloat16)
    return lax.dot_general(
        ctx, wot, (((1,), (1,)), ((), ())),
        preferred_element_type=jnp.float32,
    )


def _body(x_ref, w_ref, k_ref, v_ref, out_ref, comm_ref, send_sems, recv_sems):
    my = lax.axis_index("i")
    right = lax.rem(my + 1, N_DEV)
    left = lax.rem(my + N_DEV - 1, N_DEV)
    groups = [lax.rem(my + off, N_DEV) for off in _GROUP_OFF]

    barrier_sem = pltpu.get_barrier_semaphore()
    for nbr in (left, right):
        pl.semaphore_signal(
            barrier_sem, inc=1,
            device_id=(nbr,), device_id_type=pl.DeviceIdType.MESH,
        )
    pl.semaphore_wait(barrier_sem, 2)

    xb = x_ref[...].astype(jnp.bfloat16)

    def rdma(src_ref, dst_slot, send_idx, target):
        return pltpu.make_async_remote_copy(
            src_ref=src_ref,
            dst_ref=comm_ref.at[dst_slot],
            send_sem=send_sems.at[send_idx],
            recv_sem=recv_sems.at[dst_slot],
            device_id=(target,),
            device_id_type=pl.DeviceIdType.MESH,
        )

    cw0a = rdma(w_ref.at[0], 0, 0, right)
    cw0b = rdma(w_ref.at[1], 1, 1, right)
    ccw0a = rdma(w_ref.at[1], 3, 2, left)
    ccw0b = rdma(w_ref.at[0], 2, 3, left)
    cw0a.start()
    ccw0a.start()
    cw0b.start()
    ccw0b.start()

    acc = _unit_contrib(xb, w_ref[0], k_ref, v_ref, groups[0], 0)

    cw0a.wait_recv()
    fwd_cw = rdma(comm_ref.at[0], 4, 4, right)
    fwd_cw.start()
    ccw0a.wait_recv()
    fwd_ccw = rdma(comm_ref.at[3], 5, 5, left)
    fwd_ccw.start()

    acc = acc + _unit_contrib(xb, w_ref[1], k_ref, v_ref, groups[0], 1)
    acc = acc + _unit_contrib(xb, comm_ref[0], k_ref, v_ref, groups[1], 2)
    acc = acc + _unit_contrib(xb, comm_ref[3], k_ref, v_ref, groups[2], 5)

    cw0b.wait_recv()
    acc = acc + _unit_contrib(xb, comm_ref[1], k_ref, v_ref, groups[1], 3)
    ccw0b.wait_recv()
    acc = acc + _unit_contrib(xb, comm_ref[2], k_ref, v_ref, groups[2], 4)

    fwd_cw.wait_recv()
    acc = acc + _unit_contrib(xb, comm_ref[4], k_ref, v_ref, groups[3], 6)
    fwd_ccw.wait_recv()
    acc = acc + _unit_contrib(xb, comm_ref[5], k_ref, v_ref, groups[3], 7)

    for d in (cw0a, cw0b, ccw0a, ccw0b, fwd_cw, fwd_ccw):
        d.wait_send()

    out_ref[...] = acc


def kernel(x, Wq, K_ext, V_ext, Wo):
    my = lax.axis_index("i")

    kb = lax.dynamic_slice_in_dim(K_ext, my * B_PER, B_PER, axis=0)
    vb = lax.dynamic_slice_in_dim(V_ext, my * B_PER, B_PER, axis=0)
    kt = jnp.transpose(kb, (0, 2, 3, 1))
    vt = jnp.transpose(vb, (0, 2, 3, 1))

    xf = x.reshape(B_PER * SQ, D_MODEL)
    wq_h = Wq.astype(jnp.bfloat16).reshape(D_MODEL, HG // HPP, W_HALF).transpose(1, 0, 2)
    wot_h = Wo.T.astype(jnp.bfloat16).reshape(D_MODEL, HG // HPP, W_HALF).transpose(1, 0, 2)
    pack_bf = jnp.concatenate([wq_h, wot_h], axis=2)
    pack_u16 = lax.bitcast_convert_type(pack_bf, jnp.uint16)
    pack_u32 = lax.bitcast_convert_type(
        pack_u16.reshape(2, D_MODEL // 2, 2, D_HEADS).transpose(0, 1, 3, 2),
        jnp.uint32,
    )

    out = pl.pallas_call(
        _body,
        out_shape=jax.ShapeDtypeStruct((B_PER * SQ, D_MODEL), jnp.float32),
        in_specs=[
            pl.BlockSpec(memory_space=pltpu.VMEM),
            pl.BlockSpec(memory_space=pltpu.VMEM),
            pl.BlockSpec(memory_space=pltpu.VMEM),
            pl.BlockSpec(memory_space=pltpu.VMEM),
        ],
        out_specs=pl.BlockSpec(memory_space=pltpu.VMEM),
        scratch_shapes=[
            pltpu.VMEM((6, D_MODEL // 2, D_HEADS), jnp.uint32),
            pltpu.SemaphoreType.DMA((6,)),
            pltpu.SemaphoreType.DMA((6,)),
        ],
        compiler_params=pltpu.CompilerParams(collective_id=0),
    )(xf, pack_u32, kt, vt)

    return out.reshape(B_PER, SQ, D_MODEL)
